# Initial kernel scaffold; baseline (speedup 1.0000x reference)
import jax
import jax.numpy as jnp
from jax.experimental import pallas as pl


def kernel(feats, s_inv, feats_idx):
    raise NotImplementedError("write your pallas kernel here")


# speedup vs baseline: 1.2576x; 1.2576x over previous
import functools

import jax
import jax.numpy as jnp
from jax import lax
from jax.experimental import pallas as pl
from jax.experimental.pallas import tpu as pltpu
from jax.experimental.pallas import tpu_sc as plsc

N = 1000000
RHO = 0.99
ALPHA = 0.5
TEMP = 0.5
B = 4096
EPS = 1e-6

NC = 2
NS = 16
NW = NC * NS
CHUNK = B // NW


def _sc_gather_body(s_inv_hbm, idx_hbm, out_hbm, idx_v, s_v, sem):
    wid = lax.axis_index("s") * NC + lax.axis_index("c")
    base = wid * CHUNK
    pltpu.sync_copy(idx_hbm.at[pl.ds(base, CHUNK)], idx_v)
    pltpu.async_copy(s_inv_hbm.at[idx_v], s_v, sem).wait()
    pltpu.sync_copy(s_v, out_hbm.at[pl.ds(base, CHUNK)])


def _sc_scatter_body(s_ref, idx_hbm, vals_hbm, idx_v, v_v, sem):
    wid = lax.axis_index("s") * NC + lax.axis_index("c")
    base = wid * CHUNK
    pltpu.sync_copy(idx_hbm.at[pl.ds(base, CHUNK)], idx_v)
    pltpu.sync_copy(vals_hbm.at[pl.ds(base, CHUNK)], v_v)
    pltpu.async_copy(v_v, s_ref.at[idx_v], sem).wait()


def _dense_body(f_ref, sg_ref, loss_ref, snew_ref):
    f = f_ref[...]
    norm = jnp.maximum(jnp.sqrt(jnp.sum(f * f, axis=1, keepdims=True)), 1e-12)
    fn = f / norm
    an = fn[:B]
    bn = fn[B:]
    bro = pltpu.roll(bn, B - 1, 0)
    aro = pltpu.roll(an, B - 1, 0)
    d2aa = jnp.sum((an - bn + EPS) ** 2, axis=1, keepdims=True)
    d2bb = jnp.sum((bn - an + EPS) ** 2, axis=1, keepdims=True)
    d2ra = jnp.sum((an - bro + EPS) ** 2, axis=1, keepdims=True)
    d2rb = jnp.sum((bn - aro + EPS) ** 2, axis=1, keepdims=True)
    inv2t2 = 1.0 / (2.0 * TEMP * TEMP)
    qaa = jnp.exp(-inv2t2 * d2aa)
    qab = jnp.exp(-inv2t2 * d2bb)
    qra = jnp.exp(-inv2t2 * d2ra)
    qrb = jnp.exp(-inv2t2 * d2rb)
    sg = sg_ref[...]
    n2 = jnp.float32(N) * jnp.float32(N)
    v4 = (ALPHA * 0.5) * (qaa + qab) + ((1.0 - ALPHA) * 0.5) * (qra + qrb)
    snew_ref[...] = RHO * sg + (1.0 - RHO) * n2 * v4
    attr_sum = inv2t2 * jnp.sum(d2aa + d2bb)
    rep_sum = n2 * jnp.sum((qra + qrb) / sg)
    loss_ref[0, 0] = 0.5 * (attr_sum + rep_sum) / jnp.float32(B)


@functools.cache
def _build():
    mesh = plsc.VectorSubcoreMesh(
        core_axis_name="c", subcore_axis_name="s", num_cores=NC, num_subcores=NS
    )
    sc_gather = pl.kernel(
        _sc_gather_body,
        out_type=jax.ShapeDtypeStruct((B,), jnp.float32),
        mesh=mesh,
        scratch_types=[
            pltpu.VMEM((CHUNK,), jnp.int32),
            pltpu.VMEM((CHUNK,), jnp.float32),
            pltpu.SemaphoreType.DMA,
        ],
    )
    sc_scatter = pl.kernel(
        _sc_scatter_body,
        out_type=(),
        mesh=mesh,
        scratch_types=[
            pltpu.VMEM((CHUNK,), jnp.int32),
            pltpu.VMEM((CHUNK,), jnp.float32),
            pltpu.SemaphoreType.DMA,
        ],
    )
    dense = pl.pallas_call(
        _dense_body,
        out_shape=[
            jax.ShapeDtypeStruct((1, 1), jnp.float32),
            jax.ShapeDtypeStruct((B, 1), jnp.float32),
        ],
        in_specs=[
            pl.BlockSpec(memory_space=pltpu.VMEM),
            pl.BlockSpec(memory_space=pltpu.VMEM),
        ],
        out_specs=[
            pl.BlockSpec(memory_space=pltpu.SMEM),
            pl.BlockSpec(memory_space=pltpu.VMEM),
        ],
    )
    return sc_gather, sc_scatter, dense


def kernel(feats, s_inv, feats_idx):
    sc_gather, sc_scatter, dense = _build()
    idx = feats_idx.astype(jnp.int32)
    s_gather = sc_gather(s_inv, idx)
    loss2d, snew = dense(feats, s_gather.reshape(B, 1))
    s_ref = jax.new_ref(s_inv)
    sc_scatter(s_ref, idx, snew.reshape(B))
    new_s_inv = s_ref[...]
    return loss2d[0, 0], new_s_inv
